# Initial kernel scaffold; baseline (speedup 1.0000x reference)
import jax
import jax.numpy as jnp
from jax.experimental import pallas as pl


def kernel(logits, hyp):
    raise NotImplementedError("write your pallas kernel here")


# speedup vs baseline: 3.5029x; 3.5029x over previous
import functools

import jax
import jax.numpy as jnp
from jax import lax
from jax.experimental import pallas as pl
from jax.experimental.pallas import tpu as pltpu


def _body(logits_ref, hyp_ref, out_ref, *, nt):
    t = pl.program_id(1)
    x = logits_ref[0]
    h = hyp_ref[0, 0]
    tb, v = x.shape

    col = lax.broadcasted_iota(jnp.int32, (tb, v), 1)
    g = jnp.sum(jnp.where(col == h, x, 0.0), axis=1, keepdims=True)

    m = jnp.max(x, axis=1, keepdims=True)
    s = jnp.sum(jnp.exp(x - m), axis=1, keepdims=True)
    lse = m + jnp.log(s)

    partial = jnp.sum(g - lse).reshape(1, 1)

    @pl.when(t == 0)
    def _():
        out_ref[0] = jnp.zeros((1, 1), jnp.float32)

    out_ref[0] += partial


def kernel(logits, hyp):
    b, t, v = logits.shape
    tb = 256
    nt = t // tb
    hyp4 = hyp.astype(jnp.int32).reshape(b, nt, tb, 1)

    out = pl.pallas_call(
        functools.partial(_body, nt=nt),
        grid=(b, nt),
        in_specs=[
            pl.BlockSpec((1, tb, v), lambda i, j: (i, j, 0)),
            pl.BlockSpec((1, 1, tb, 1), lambda i, j: (i, j, 0, 0)),
        ],
        out_specs=pl.BlockSpec((1, 1, 1), lambda i, j: (i, 0, 0)),
        out_shape=jax.ShapeDtypeStruct((b, 1, 1), jnp.float32),
        compiler_params=pltpu.CompilerParams(
            dimension_semantics=("arbitrary", "arbitrary"),
        ),
    )(logits, hyp4)
    return out[:, 0, 0]
